# baseline (speedup 1.0000x reference)
import jax
import jax.numpy as jnp
from jax.experimental import pallas as pl
from jax.experimental.pallas import tpu as pltpu

E = 8
DIM = 1024
FFN = 2048
OUT = 1024
EPS = 1e-05


def _gate_body(x_ref, wg_ref, out_ref, outt_ref):
    xc = x_ref[0]
    out_ref[0] = jax.lax.dot_general(
        xc, wg_ref[...], (((1,), (1,)), ((), ())),
        preferred_element_type=jnp.float32)
    outt_ref[0] = jax.lax.dot_general(
        wg_ref[...], xc, (((1,), (1,)), ((), ())),
        preferred_element_type=jnp.float32)


def _ffn_body(x_hbm, xg_ref, idx_ref, w1_ref, b1_ref, lng_ref, lnb_ref,
              w2_ref, b2_ref, out_ref, xbuf, sem):
    e = pl.program_id(0)
    b = pl.program_id(1)
    B = xbuf.shape[0]
    S = xg_ref.shape[1]
    k = idx_ref.shape[-1]

    @pl.when((e == 0) & (b == 0))
    def _():
        for j in range(B):
            pltpu.make_async_copy(x_hbm.at[j], xbuf.at[j], sem.at[j]).start()
        pltpu.make_async_copy(x_hbm.at[0], xbuf.at[0], sem.at[0]).wait()

    for j in range(1, B):
        @pl.when((e == 0) & (b == j))
        def _(j=j):
            pltpu.make_async_copy(x_hbm.at[j], xbuf.at[j], sem.at[j]).wait()

    idx = idx_ref[0, 0, 0]
    sio = jax.lax.broadcasted_iota(jnp.int32, (k, S), 1)
    ohb = sio == idx[:, None]
    oh = ohb.astype(jnp.bfloat16)
    xs = jnp.dot(oh, xbuf[b].astype(jnp.bfloat16),
                 preferred_element_type=jnp.float32).astype(jnp.bfloat16)

    xg = xg_ref[0]
    m = jnp.max(xg, axis=-1, keepdims=True)
    ex = jnp.exp(xg - m)
    gs = ex / jnp.sum(ex, axis=-1, keepdims=True)
    sel = (jax.lax.broadcasted_iota(jnp.int32, (xg.shape[1], 1), 0) == e
           ).astype(jnp.float32)
    gs_col = jnp.dot(gs, sel, preferred_element_type=jnp.float32)
    g = jnp.dot(ohb.astype(jnp.float32), gs_col,
                preferred_element_type=jnp.float32)[:, 0]

    h = jax.lax.dot_general(
        xs, w1_ref[0].astype(jnp.bfloat16),
        (((1,), (1,)), ((), ())),
        preferred_element_type=jnp.float32) + b1_ref[0, 0][None, :]
    h = h * 0.5 * (1.0 + jax.lax.erf(h * (2.0 ** -0.5)))

    mu = jnp.mean(h, axis=-1, keepdims=True)
    var = jnp.mean((h - mu) ** 2, axis=-1, keepdims=True)
    h = (h - mu) / jnp.sqrt(var + EPS) * lng_ref[0, 0][None, :] + lnb_ref[0, 0][None, :]

    y = jax.lax.dot_general(
        h.astype(jnp.bfloat16), w2_ref[0].astype(jnp.bfloat16),
        (((1,), (1,)), ((), ())),
        preferred_element_type=jnp.float32) + b2_ref[0, 0][None, :]
    out_ref[0, 0] = (y * g[:, None]).astype(jnp.bfloat16)


def _scatter_body(y_ref, idx_ref, out_ref):
    S = out_ref.shape[1]
    ek = idx_ref.shape[-1]
    idx = idx_ref[0, 0]
    sio = jax.lax.broadcasted_iota(jnp.int32, (ek, S), 1)
    oh = (sio == idx[:, None]).astype(jnp.bfloat16)
    out_ref[0] = jax.lax.dot_general(
        oh, y_ref[0], (((0,), (0,)), ((), ())),
        preferred_element_type=jnp.float32)


def kernel(x, Wg, W1, b1, ln_g, ln_b, W2, b2):
    B, S, _ = x.shape
    k = max(1, int(S // E))

    SC = S // 4
    x_gated, xg_t = pl.pallas_call(
        _gate_body,
        grid=(B, 4),
        in_specs=[
            pl.BlockSpec((1, SC, DIM), lambda b, i: (b, i, 0)),
            pl.BlockSpec((E, DIM), lambda b, i: (0, 0)),
        ],
        out_specs=[
            pl.BlockSpec((1, SC, E), lambda b, i: (b, i, 0)),
            pl.BlockSpec((1, E, SC), lambda b, i: (b, 0, i)),
        ],
        out_shape=[
            jax.ShapeDtypeStruct((B, S, E), jnp.float32),
            jax.ShapeDtypeStruct((B, E, S), jnp.float32),
        ],
    )(x, Wg)

    _, idx = jax.lax.top_k(xg_t, k)
    idx4 = idx.astype(jnp.int32).reshape(B, E, 1, k)

    y_sel = pl.pallas_call(
        _ffn_body,
        grid=(E, B),
        in_specs=[
            pl.BlockSpec(memory_space=pl.ANY),
            pl.BlockSpec((1, S, E), lambda e, b: (b, 0, 0)),
            pl.BlockSpec((1, 1, 1, k), lambda e, b: (b, e, 0, 0)),
            pl.BlockSpec((1, FFN, DIM), lambda e, b: (e, 0, 0)),
            pl.BlockSpec((1, 1, FFN), lambda e, b: (e, 0, 0)),
            pl.BlockSpec((1, 1, FFN), lambda e, b: (e, 0, 0)),
            pl.BlockSpec((1, 1, FFN), lambda e, b: (e, 0, 0)),
            pl.BlockSpec((1, OUT, FFN), lambda e, b: (e, 0, 0)),
            pl.BlockSpec((1, 1, OUT), lambda e, b: (e, 0, 0)),
        ],
        out_specs=pl.BlockSpec((1, 1, k, OUT), lambda e, b: (b, e, 0, 0)),
        out_shape=jax.ShapeDtypeStruct((B, E, k, OUT), jnp.bfloat16),
        scratch_shapes=[
            pltpu.VMEM((B, S, DIM), jnp.float32),
            pltpu.SemaphoreType.DMA((B,)),
        ],
    )(x, x_gated, idx4, W1, b1.reshape(E, 1, FFN), ln_g.reshape(E, 1, FFN),
      ln_b.reshape(E, 1, FFN), W2, b2.reshape(E, 1, OUT))

    moe_output = pl.pallas_call(
        _scatter_body,
        grid=(B,),
        in_specs=[
            pl.BlockSpec((1, E * k, OUT), lambda b: (b, 0, 0)),
            pl.BlockSpec((1, 1, E * k), lambda b: (b, 0, 0)),
        ],
        out_specs=pl.BlockSpec((1, S, OUT), lambda b: (b, 0, 0)),
        out_shape=jax.ShapeDtypeStruct((B, S, OUT), jnp.float32),
    )(y_sel.reshape(B, E * k, OUT), idx.astype(jnp.int32).reshape(B, 1, E * k))

    return (moe_output, x_gated)


# speedup vs baseline: 1.0115x; 1.0115x over previous
import jax
import jax.numpy as jnp
from jax.experimental import pallas as pl
from jax.experimental.pallas import tpu as pltpu

E = 8
DIM = 1024
FFN = 2048
OUT = 1024
EPS = 1e-05


def _gate_body(x_ref, wg_ref, out_ref):
    out_ref[...] = jax.lax.dot_general(
        x_ref[...], wg_ref[...],
        (((1,), (1,)), ((), ())),
        preferred_element_type=jnp.float32,
    )


def _ffn_body(x_hbm, xg_ref, idx_ref, w1_ref, b1_ref, lng_ref, lnb_ref,
              w2_ref, b2_ref, out_ref, xbuf, sem):
    e = pl.program_id(0)
    b = pl.program_id(1)
    B = xbuf.shape[0]
    S = xg_ref.shape[1]
    k = idx_ref.shape[-1]

    @pl.when((e == 0) & (b == 0))
    def _():
        for j in range(B):
            pltpu.make_async_copy(x_hbm.at[j], xbuf.at[j], sem.at[j]).start()
        pltpu.make_async_copy(x_hbm.at[0], xbuf.at[0], sem.at[0]).wait()

    for j in range(1, B):
        @pl.when((e == 0) & (b == j))
        def _(j=j):
            pltpu.make_async_copy(x_hbm.at[j], xbuf.at[j], sem.at[j]).wait()

    idx = idx_ref[0, 0, 0]
    sio = jax.lax.broadcasted_iota(jnp.int32, (k, S), 1)
    ohb = sio == idx[:, None]
    oh = ohb.astype(jnp.bfloat16)
    xs = jnp.dot(oh, xbuf[b].astype(jnp.bfloat16),
                 preferred_element_type=jnp.float32).astype(jnp.bfloat16)

    xg = xg_ref[0]
    m = jnp.max(xg, axis=-1, keepdims=True)
    ex = jnp.exp(xg - m)
    gs = ex / jnp.sum(ex, axis=-1, keepdims=True)
    sel = (jax.lax.broadcasted_iota(jnp.int32, (xg.shape[1], 1), 0) == e
           ).astype(jnp.float32)
    gs_col = jnp.dot(gs, sel, preferred_element_type=jnp.float32)
    g = jnp.dot(ohb.astype(jnp.float32), gs_col,
                preferred_element_type=jnp.float32)[:, 0]

    h = jax.lax.dot_general(
        xs, w1_ref[0].astype(jnp.bfloat16),
        (((1,), (1,)), ((), ())),
        preferred_element_type=jnp.float32) + b1_ref[0, 0][None, :]
    h = h * 0.5 * (1.0 + jax.lax.erf(h * (2.0 ** -0.5)))

    mu = jnp.mean(h, axis=-1, keepdims=True)
    var = jnp.mean((h - mu) ** 2, axis=-1, keepdims=True)
    h = (h - mu) / jnp.sqrt(var + EPS) * lng_ref[0, 0][None, :] + lnb_ref[0, 0][None, :]

    y = jax.lax.dot_general(
        h.astype(jnp.bfloat16), w2_ref[0].astype(jnp.bfloat16),
        (((1,), (1,)), ((), ())),
        preferred_element_type=jnp.float32) + b2_ref[0, 0][None, :]
    out_ref[0, 0] = (y * g[:, None]).astype(jnp.bfloat16)


def _scatter_body(y_ref, idx_ref, out_ref):
    S = out_ref.shape[1]
    ek = idx_ref.shape[-1]
    idx = idx_ref[0, 0]
    sio = jax.lax.broadcasted_iota(jnp.int32, (ek, S), 1)
    oh = (sio == idx[:, None]).astype(jnp.bfloat16)
    out_ref[0] = jax.lax.dot_general(
        oh, y_ref[0], (((0,), (0,)), ((), ())),
        preferred_element_type=jnp.float32)


def kernel(x, Wg, W1, b1, ln_g, ln_b, W2, b2):
    B, S, _ = x.shape
    k = max(1, int(S // E))

    x_gated = pl.pallas_call(
        _gate_body,
        out_shape=jax.ShapeDtypeStruct((B * S, E), jnp.float32),
    )(x.reshape(B * S, DIM), Wg).reshape(B, S, E)

    xg_t = jnp.transpose(x_gated, (0, 2, 1))
    _, idx = jax.lax.top_k(xg_t, k)
    idx4 = idx.astype(jnp.int32).reshape(B, E, 1, k)

    y_sel = pl.pallas_call(
        _ffn_body,
        grid=(E, B),
        in_specs=[
            pl.BlockSpec(memory_space=pl.ANY),
            pl.BlockSpec((1, S, E), lambda e, b: (b, 0, 0)),
            pl.BlockSpec((1, 1, 1, k), lambda e, b: (b, e, 0, 0)),
            pl.BlockSpec((1, FFN, DIM), lambda e, b: (e, 0, 0)),
            pl.BlockSpec((1, 1, FFN), lambda e, b: (e, 0, 0)),
            pl.BlockSpec((1, 1, FFN), lambda e, b: (e, 0, 0)),
            pl.BlockSpec((1, 1, FFN), lambda e, b: (e, 0, 0)),
            pl.BlockSpec((1, OUT, FFN), lambda e, b: (e, 0, 0)),
            pl.BlockSpec((1, 1, OUT), lambda e, b: (e, 0, 0)),
        ],
        out_specs=pl.BlockSpec((1, 1, k, OUT), lambda e, b: (b, e, 0, 0)),
        out_shape=jax.ShapeDtypeStruct((B, E, k, OUT), jnp.bfloat16),
        scratch_shapes=[
            pltpu.VMEM((B, S, DIM), jnp.float32),
            pltpu.SemaphoreType.DMA((B,)),
        ],
    )(x, x_gated, idx4, W1, b1.reshape(E, 1, FFN), ln_g.reshape(E, 1, FFN),
      ln_b.reshape(E, 1, FFN), W2, b2.reshape(E, 1, OUT))

    moe_output = pl.pallas_call(
        _scatter_body,
        grid=(B,),
        in_specs=[
            pl.BlockSpec((1, E * k, OUT), lambda b: (b, 0, 0)),
            pl.BlockSpec((1, 1, E * k), lambda b: (b, 0, 0)),
        ],
        out_specs=pl.BlockSpec((1, S, OUT), lambda b: (b, 0, 0)),
        out_shape=jax.ShapeDtypeStruct((B, S, OUT), jnp.float32),
    )(y_sel.reshape(B, E * k, OUT), idx.astype(jnp.int32).reshape(B, 1, E * k))

    return (moe_output, x_gated)
